# baseline (speedup 1.0000x reference)
import functools

import jax
import jax.numpy as jnp
from jax import lax
from jax.experimental import pallas as pl
from jax.experimental.pallas import tpu as pltpu
from jax.experimental.pallas import tpu_sc as plsc

N_LANES = 16
N_CORES = 2
N_SUBCORES = 16
NUM_WORKERS = N_CORES * N_SUBCORES


def _gather_add_body(per_worker, chunk, num_chunks,
                     range_hbm, params_hbm, idx_hbm, out_hbm,
                     table_sh, table_v, out_s0, out_s1,
                     idx_v0, idx_v1, rng_v0, rng_v1,
                     isem0, isem1, osem0, osem1, dsem0, dsem1, tsem):
    s_id = lax.axis_index("s")
    wid = s_id * 2 + lax.axis_index("c")
    base = wid * per_worker
    idx_v = (idx_v0, idx_v1)
    rng_v = (rng_v0, rng_v1)
    out_s = (out_s0, out_s1)
    isems = (isem0, isem1)
    osems = (osem0, osem1)
    dsems = (dsem0, dsem1)

    in_copies = {}
    ostreams = {}
    dmas_out = {}

    def issue_in(j):
        b = j & 1
        off = base + j * chunk
        in_copies[j] = (
            pltpu.async_copy(idx_hbm.at[pl.ds(off, chunk)],
                             idx_v[b], isems[b]),
            pltpu.async_copy(range_hbm.at[pl.ds(off, chunk)],
                             rng_v[b], isems[b]),
        )

    issue_in(0)

    @pl.when(s_id == 0)
    def _():
        pltpu.sync_copy(params_hbm, table_sh)

    plsc.subcore_barrier()
    tcopy = pltpu.async_copy(table_sh, table_v, tsem)

    for j in range(num_chunks):
        b = j & 1
        if j >= 1:
            ostreams.pop(j - 1).wait()
            r = wid * num_chunks + (j - 1)
            dmas_out[j - 1] = pltpu.async_copy(
                out_s[(j - 1) & 1].at[s_id], out_hbm.at[r], dsems[(j - 1) & 1])
        if j + 1 < num_chunks:
            issue_in(j + 1)
        ci, cr = in_copies.pop(j)
        ci.wait()
        cr.wait()
        if j == 0:
            tcopy.wait()

        idx_b = idx_v[b]
        rng_b = rng_v[b]

        @plsc.parallel_loop(0, chunk, step=N_LANES, unroll=8)
        def _(i):
            s = pl.ds(i, N_LANES)
            vals = plsc.load_gather(table_v, [idx_b[s]])
            rng_b[s] = rng_b[s] + vals

        if j >= 2:
            dmas_out.pop(j - 2).wait()
        ostreams[j] = pltpu.async_copy(rng_b, out_s[b].at[s_id], osems[b])

    last = num_chunks - 1
    ostreams.pop(last).wait()
    r = wid * num_chunks + last
    dmas_out[last] = pltpu.async_copy(out_s[last & 1].at[s_id],
                                      out_hbm.at[r], dsems[last & 1])
    for j in sorted(dmas_out):
        dmas_out[j].wait()


@jax.jit
def _radar_bias_add(range_km, sensor_params, contact_indices):
    n_meas = range_km.shape[0]
    n_passes = sensor_params.shape[0]
    assert n_meas % NUM_WORKERS == 0
    per_worker = n_meas // NUM_WORKERS
    chunk = 4096
    assert per_worker % chunk == 0
    num_chunks = per_worker // chunk
    total_rows = NUM_WORKERS * num_chunks

    mesh = plsc.VectorSubcoreMesh(core_axis_name="c", subcore_axis_name="s")
    body = functools.partial(_gather_add_body, per_worker, chunk, num_chunks)
    f = pl.kernel(
        body,
        out_type=jax.ShapeDtypeStruct((total_rows, chunk), jnp.float32),
        mesh=mesh,
        compiler_params=pltpu.CompilerParams(needs_layout_passes=False),
        scratch_types=[
            pltpu.VMEM_SHARED((n_passes,), jnp.float32),
            pltpu.VMEM((n_passes,), jnp.float32),
            pltpu.VMEM_SHARED((N_SUBCORES, chunk), jnp.float32),
            pltpu.VMEM_SHARED((N_SUBCORES, chunk), jnp.float32),
            pltpu.VMEM((chunk,), jnp.int32),
            pltpu.VMEM((chunk,), jnp.int32),
            pltpu.VMEM((chunk,), jnp.float32),
            pltpu.VMEM((chunk,), jnp.float32),
            pltpu.SemaphoreType.DMA,
            pltpu.SemaphoreType.DMA,
            pltpu.SemaphoreType.DMA,
            pltpu.SemaphoreType.DMA,
            pltpu.SemaphoreType.DMA,
            pltpu.SemaphoreType.DMA,
            pltpu.SemaphoreType.DMA,
        ],
    )
    out = f(range_km, sensor_params, contact_indices)
    return out.reshape(n_meas)


def kernel(range_km, sensor_params, contact_indices):
    idx = contact_indices.astype(jnp.int32)
    return _radar_bias_add(range_km, sensor_params, idx)


# speedup vs baseline: 1.3452x; 1.3452x over previous
import functools

import jax
import jax.numpy as jnp
from jax import lax
from jax.experimental import pallas as pl
from jax.experimental.pallas import tpu as pltpu
from jax.experimental.pallas import tpu_sc as plsc

N_LANES = 16
NUM_WORKERS = 32


def _gather_add_body(per_worker, chunk, num_chunks,
                     range_hbm, params_hbm, idx_hbm, out_hbm,
                     table_sh, table_v,
                     idx_v0, idx_v1, rng_v0, rng_v1, res_v0, res_v1,
                     isem0, isem1, osem0, osem1, tsem):
    s_id = lax.axis_index("s")
    wid = s_id * 2 + lax.axis_index("c")
    base = wid * per_worker
    idx_v = (idx_v0, idx_v1)
    rng_v = (rng_v0, rng_v1)
    res_v = (res_v0, res_v1)
    isems = (isem0, isem1)
    osems = (osem0, osem1)

    in_copies = {}
    out_copies = {}

    def issue_in(j):
        b = j & 1
        off = base + j * chunk
        in_copies[j] = (
            pltpu.async_copy(idx_hbm.at[pl.ds(off, chunk)],
                             idx_v[b], isems[b]),
            pltpu.async_copy(range_hbm.at[pl.ds(off, chunk)],
                             rng_v[b], isems[b]),
        )

    issue_in(0)

    @pl.when(s_id == 0)
    def _():
        pltpu.sync_copy(params_hbm, table_sh)

    plsc.subcore_barrier()
    tcopy = pltpu.async_copy(table_sh, table_v, tsem)

    for j in range(num_chunks):
        b = j & 1
        if j + 1 < num_chunks:
            issue_in(j + 1)
        ci, cr = in_copies.pop(j)
        ci.wait()
        cr.wait()
        if j == 0:
            tcopy.wait()
        if j >= 2:
            out_copies.pop(j - 2).wait()

        idx_b = idx_v[b]
        rng_b = rng_v[b]
        res_b = res_v[b]

        @plsc.parallel_loop(0, chunk, step=N_LANES, unroll=8)
        def _(i):
            s = pl.ds(i, N_LANES)
            vals = plsc.load_gather(table_v, [idx_b[s]])
            res_b[s] = rng_b[s] + vals

        out_copies[j] = pltpu.async_copy(
            res_v[b], out_hbm.at[pl.ds(base + j * chunk, chunk)], osems[b])

    for j in sorted(out_copies):
        out_copies[j].wait()


@jax.jit
def _radar_bias_add(range_km, sensor_params, contact_indices):
    n_meas = range_km.shape[0]
    n_passes = sensor_params.shape[0]
    assert n_meas % NUM_WORKERS == 0
    per_worker = n_meas // NUM_WORKERS
    chunk = 4096
    assert per_worker % chunk == 0
    num_chunks = per_worker // chunk

    mesh = plsc.VectorSubcoreMesh(core_axis_name="c", subcore_axis_name="s")
    body = functools.partial(_gather_add_body, per_worker, chunk, num_chunks)
    f = pl.kernel(
        body,
        out_type=jax.ShapeDtypeStruct((n_meas,), jnp.float32),
        mesh=mesh,
        compiler_params=pltpu.CompilerParams(needs_layout_passes=False),
        scratch_types=[
            pltpu.VMEM_SHARED((n_passes,), jnp.float32),
            pltpu.VMEM((n_passes,), jnp.float32),
            pltpu.VMEM((chunk,), jnp.int32),
            pltpu.VMEM((chunk,), jnp.int32),
            pltpu.VMEM((chunk,), jnp.float32),
            pltpu.VMEM((chunk,), jnp.float32),
            pltpu.VMEM((chunk,), jnp.float32),
            pltpu.VMEM((chunk,), jnp.float32),
            pltpu.SemaphoreType.DMA,
            pltpu.SemaphoreType.DMA,
            pltpu.SemaphoreType.DMA,
            pltpu.SemaphoreType.DMA,
            pltpu.SemaphoreType.DMA,
        ],
    )
    return f(range_km, sensor_params, contact_indices)


def kernel(range_km, sensor_params, contact_indices):
    idx = contact_indices.astype(jnp.int32)
    return _radar_bias_add(range_km, sensor_params, idx)
